# baseline (speedup 1.0000x reference)
import functools

import jax
import jax.numpy as jnp
from jax import lax
from jax.experimental import pallas as pl
from jax.experimental.pallas import tpu as pltpu

B, P = 8, 2048
LATENT = 64
GRID = 50
K_NBR = 64


def _linear(x, w, b):
    return x @ w.T + b


def _fps(pos_g, n_sample):
    N = pos_g.shape[0]

    def body(carry, _):
        mind, last = carry
        d = jnp.sum((pos_g - pos_g[last]) ** 2, axis=-1)
        mind = jnp.minimum(mind, d)
        nxt = jnp.argmax(mind).astype(jnp.int32)
        return (mind, nxt), nxt

    init = (jnp.full((N,), 1e30, dtype=pos_g.dtype), jnp.asarray(0, dtype=jnp.int32))
    _, rest = lax.scan(body, init, None, length=n_sample - 1)
    return jnp.concatenate([jnp.zeros((1,), dtype=jnp.int32), rest])


def _sa(x, pos_g, n_sample, r, w1, b1, w2, b2):
    idx = _fps(pos_g, n_sample)
    q = pos_g[idx]
    d2 = jnp.sum((q[:, None, :] - pos_g[None, :, :]) ** 2, axis=-1)
    neg = jnp.where(d2 <= r * r, -d2, -jnp.inf)
    vals, nbr = lax.top_k(neg, K_NBR)
    valid = vals > -1e30
    rel = pos_g[nbr] - q[:, None, :]
    feat = rel if x is None else jnp.concatenate([x[nbr], rel], axis=-1)
    h = jax.nn.relu(_linear(feat, w1, b1))
    h = _linear(h, w2, b2)
    h = jnp.where(valid[..., None], h, -1e9)
    out = jnp.max(h, axis=1)
    out = jnp.where(out <= -1e8, 0.0, out)
    return out, q


def _encode_graph(pos_g, params):
    x, q = _sa(None, pos_g, P // 2, 0.2, params['sa1_w1'], params['sa1_b1'], params['sa1_w2'], params['sa1_b2'])
    x, q = _sa(x, q, P // 8, 0.4, params['sa2_w1'], params['sa2_b1'], params['sa2_w2'], params['sa2_b2'])
    x, q = _sa(x, q, P // 32, 0.8, params['sa3_w1'], params['sa3_b1'], params['sa3_w2'], params['sa3_b2'])
    return jnp.max(x, axis=0)


def _dec_kernel(z_ref, tgt_ref, gx_ref, gy_ref,
                f1w1z_ref, f1w1g_ref, f1b1_ref, f1w2_ref, f1b2_ref, f1w3_ref, f1b3_ref,
                f2w1z_ref, f2w1x_ref, f2b1_ref, f2w2_ref, f2b2_ref, f2w3_ref, f2b3_ref,
                recon_ref, s1_ref, s2_ref):
    z = z_ref[0]
    gx = gx_ref[...]
    gy = gy_ref[...]

    zh = z @ f1w1z_ref[...].T + f1b1_ref[...]
    h = zh + gx * f1w1g_ref[0:1, :] + gy * f1w1g_ref[1:2, :]
    h = jnp.maximum(h, 0.0)
    h = jnp.maximum(h @ f1w2_ref[...].T + f1b2_ref[...], 0.0)
    x1 = h @ f1w3_ref[...].T + f1b3_ref[...]

    zh2 = z @ f2w1z_ref[...].T + f2b1_ref[...]
    h = zh2 + (x1[:, 0:1] * f2w1x_ref[0:1, :]
               + x1[:, 1:2] * f2w1x_ref[1:2, :]
               + x1[:, 2:3] * f2w1x_ref[2:3, :])
    h = jnp.maximum(h, 0.0)
    h = jnp.maximum(h @ f2w2_ref[...].T + f2b2_ref[...], 0.0)
    x2 = h @ f2w3_ref[...].T + f2b3_ref[...]

    recon_ref[0] = x2

    tgt = tgt_ref[0]
    a2 = jnp.sum(x2 * x2, axis=1, keepdims=True)
    b2 = jnp.sum(tgt * tgt, axis=0, keepdims=True)
    dot = (x2[:, 0:1] * tgt[0:1, :]
           + x2[:, 1:2] * tgt[1:2, :]
           + x2[:, 2:3] * tgt[2:3, :])
    d2 = a2 + b2 - 2.0 * dot
    d = jnp.sqrt(jnp.maximum(d2, 0.0) + 1e-12)
    s1_ref[0] = jnp.sum(jnp.min(d, axis=1)).reshape(1, 1)
    s2_ref[0] = jnp.sum(jnp.min(d, axis=0)).reshape(1, 1)


def _decode_chamfer(z, pos_pad, params):
    G2 = GRID * GRID
    xs = jnp.linspace(-0.3, 0.3, GRID)
    gxm, gym = jnp.meshgrid(xs, xs, indexing='ij')
    gx = gxm.ravel()[:, None].astype(jnp.float32)
    gy = gym.ravel()[:, None].astype(jnp.float32)

    f1w1 = params['f1_w1']
    f2w1 = params['f2_w1']
    f1w3 = jnp.zeros((8, 512), jnp.float32).at[:3].set(params['f1_w3'])
    f1b3 = jnp.zeros((1, 8), jnp.float32).at[0, :3].set(params['f1_b3'])
    f2w3 = jnp.zeros((8, 512), jnp.float32).at[:3].set(params['f2_w3'])
    f2b3 = jnp.zeros((1, 8), jnp.float32).at[0, :3].set(params['f2_b3'])

    bcast = lambda b: (0, 0)
    recon, s1, s2 = pl.pallas_call(
        _dec_kernel,
        grid=(B,),
        in_specs=[
            pl.BlockSpec((1, 1, LATENT), lambda b: (b, 0, 0)),
            pl.BlockSpec((1, 8, P), lambda b: (b, 0, 0)),
            pl.BlockSpec((G2, 1), bcast),
            pl.BlockSpec((G2, 1), bcast),
            pl.BlockSpec((512, LATENT), bcast),
            pl.BlockSpec((2, 512), bcast),
            pl.BlockSpec((1, 512), bcast),
            pl.BlockSpec((512, 512), bcast),
            pl.BlockSpec((1, 512), bcast),
            pl.BlockSpec((8, 512), bcast),
            pl.BlockSpec((1, 8), bcast),
            pl.BlockSpec((512, LATENT), bcast),
            pl.BlockSpec((3, 512), bcast),
            pl.BlockSpec((1, 512), bcast),
            pl.BlockSpec((512, 512), bcast),
            pl.BlockSpec((1, 512), bcast),
            pl.BlockSpec((8, 512), bcast),
            pl.BlockSpec((1, 8), bcast),
        ],
        out_specs=[
            pl.BlockSpec((1, G2, 8), lambda b: (b, 0, 0)),
            pl.BlockSpec((1, 1, 1), lambda b: (b, 0, 0)),
            pl.BlockSpec((1, 1, 1), lambda b: (b, 0, 0)),
        ],
        out_shape=[
            jax.ShapeDtypeStruct((B, G2, 8), jnp.float32),
            jax.ShapeDtypeStruct((B, 1, 1), jnp.float32),
            jax.ShapeDtypeStruct((B, 1, 1), jnp.float32),
        ],
    )(
        z[:, None, :], pos_pad, gx, gy,
        f1w1[:, :LATENT], f1w1[:, LATENT:].T, params['f1_b1'][None, :],
        params['f1_w2'], params['f1_b2'][None, :], f1w3, f1b3,
        f2w1[:, :LATENT], f2w1[:, LATENT:].T, params['f2_b1'][None, :],
        params['f2_w2'], params['f2_b2'][None, :], f2w3, f2b3,
    )
    recon = recon[:, :, :3]
    chamfer = jnp.sum(s1) / (B * G2) + jnp.sum(s2) / (B * P)
    return recon, chamfer


def kernel(pos, params, batch):
    pos_b = pos.reshape(B, P, 3)
    pooled = jax.vmap(_encode_graph, in_axes=(0, None))(pos_b, params)
    mu = _linear(pooled, params['mu_w'], params['mu_b'])
    logvar = _linear(pooled, params['lv_w'], params['lv_b'])
    std = jnp.exp(0.5 * logvar)
    eps = jax.random.normal(jax.random.key(42), mu.shape, dtype=mu.dtype)
    z = mu + std * eps

    pos_pad = jnp.zeros((B, 8, P), jnp.float32).at[:, :3, :].set(
        jnp.transpose(pos_b, (0, 2, 1)))
    recon, chamfer = _decode_chamfer(z, pos_pad, params)

    kl = -0.5 * jnp.mean(jnp.sum(1.0 + logvar - mu ** 2 - jnp.exp(logvar), axis=-1))
    loss = chamfer + 0.001 * kl
    return loss, chamfer, kl, mu, recon


# speedup vs baseline: 1.4752x; 1.4752x over previous
import functools

import jax
import jax.numpy as jnp
from jax import lax
from jax.experimental import pallas as pl
from jax.experimental.pallas import tpu as pltpu

B, P = 8, 2048
LATENT = 64
GRID = 50
K_NBR = 64


def _linear(x, w, b):
    return x @ w.T + b


def _fps_level(x, y, z, S, qref):
    N = x.shape[1]
    iota = lax.broadcasted_iota(jnp.int32, (B, N), 1)

    def emit(s, sel):
        xl = jnp.sum(jnp.where(sel, x, 0.0), axis=1, keepdims=True)
        yl = jnp.sum(jnp.where(sel, y, 0.0), axis=1, keepdims=True)
        zl = jnp.sum(jnp.where(sel, z, 0.0), axis=1, keepdims=True)
        qref[pl.ds(s, 1), :, 0:1] = xl[None]
        qref[pl.ds(s, 1), :, 1:2] = yl[None]
        qref[pl.ds(s, 1), :, 2:3] = zl[None]
        return xl, yl, zl

    def step(s, carry):
        mind, last = carry
        xl, yl, zl = emit(s, iota == last)
        dx = x - xl
        dy = y - yl
        dz = z - zl
        d = dx * dx + dy * dy + dz * dz
        mind = jnp.minimum(mind, d)
        m = jnp.max(mind, axis=1, keepdims=True)
        nxt = jnp.min(jnp.where(mind == m, iota, N), axis=1, keepdims=True)
        return mind, nxt

    init = (jnp.full((B, N), 1e30, dtype=jnp.float32),
            jnp.zeros((B, 1), dtype=jnp.int32))
    _, last = lax.fori_loop(0, S - 1, step, init)
    emit(S - 1, iota == last)


def _read_t(qref):
    q = qref[...]
    return (jnp.transpose(q[:, :, 0], (1, 0)),
            jnp.transpose(q[:, :, 1], (1, 0)),
            jnp.transpose(q[:, :, 2], (1, 0)))


def _fps_kernel(pos_ref, q1_ref, q2_ref, q3_ref):
    x, y, z = pos_ref[0], pos_ref[1], pos_ref[2]
    _fps_level(x, y, z, P // 2, q1_ref)
    x1, y1, z1 = _read_t(q1_ref)
    _fps_level(x1, y1, z1, P // 8, q2_ref)
    x2, y2, z2 = _read_t(q2_ref)
    _fps_level(x2, y2, z2, P // 32, q3_ref)


def _fps_all(pos_t):
    return pl.pallas_call(
        _fps_kernel,
        out_shape=[
            jax.ShapeDtypeStruct((P // 2, B, 3), jnp.float32),
            jax.ShapeDtypeStruct((P // 8, B, 3), jnp.float32),
            jax.ShapeDtypeStruct((P // 32, B, 3), jnp.float32),
        ],
    )(pos_t)


def _sa(x, pos_g, q, r, w1, b1, w2, b2):
    d2 = jnp.sum((q[:, None, :] - pos_g[None, :, :]) ** 2, axis=-1)
    neg = jnp.where(d2 <= r * r, -d2, -jnp.inf)
    vals, nbr = lax.top_k(neg, K_NBR)
    valid = vals > -1e30
    rel = pos_g[nbr] - q[:, None, :]
    feat = rel if x is None else jnp.concatenate([x[nbr], rel], axis=-1)
    h = jax.nn.relu(_linear(feat, w1, b1))
    h = _linear(h, w2, b2)
    h = jnp.where(valid[..., None], h, -1e9)
    out = jnp.max(h, axis=1)
    out = jnp.where(out <= -1e8, 0.0, out)
    return out


def _encode_graph(pos_g, q1, q2, q3, params):
    x = _sa(None, pos_g, q1, 0.2, params['sa1_w1'], params['sa1_b1'], params['sa1_w2'], params['sa1_b2'])
    x = _sa(x, q1, q2, 0.4, params['sa2_w1'], params['sa2_b1'], params['sa2_w2'], params['sa2_b2'])
    x = _sa(x, q2, q3, 0.8, params['sa3_w1'], params['sa3_b1'], params['sa3_w2'], params['sa3_b2'])
    return jnp.max(x, axis=0)


def _dec_kernel(z_ref, tgt_ref, gx_ref, gy_ref,
                f1w1z_ref, f1w1g_ref, f1b1_ref, f1w2_ref, f1b2_ref, f1w3_ref, f1b3_ref,
                f2w1z_ref, f2w1x_ref, f2b1_ref, f2w2_ref, f2b2_ref, f2w3_ref, f2b3_ref,
                recon_ref, s1_ref, s2_ref):
    z = z_ref[0]
    gx = gx_ref[...]
    gy = gy_ref[...]

    zh = z @ f1w1z_ref[...].T + f1b1_ref[...]
    h = zh + gx * f1w1g_ref[0:1, :] + gy * f1w1g_ref[1:2, :]
    h = jnp.maximum(h, 0.0)
    h = jnp.maximum(h @ f1w2_ref[...].T + f1b2_ref[...], 0.0)
    x1 = h @ f1w3_ref[...].T + f1b3_ref[...]

    zh2 = z @ f2w1z_ref[...].T + f2b1_ref[...]
    h = zh2 + (x1[:, 0:1] * f2w1x_ref[0:1, :]
               + x1[:, 1:2] * f2w1x_ref[1:2, :]
               + x1[:, 2:3] * f2w1x_ref[2:3, :])
    h = jnp.maximum(h, 0.0)
    h = jnp.maximum(h @ f2w2_ref[...].T + f2b2_ref[...], 0.0)
    x2 = h @ f2w3_ref[...].T + f2b3_ref[...]

    recon_ref[0] = x2

    tgt = tgt_ref[0]
    a2 = jnp.sum(x2 * x2, axis=1, keepdims=True)
    b2 = jnp.sum(tgt * tgt, axis=0, keepdims=True)
    dot = (x2[:, 0:1] * tgt[0:1, :]
           + x2[:, 1:2] * tgt[1:2, :]
           + x2[:, 2:3] * tgt[2:3, :])
    d2 = a2 + b2 - 2.0 * dot
    d = jnp.sqrt(jnp.maximum(d2, 0.0) + 1e-12)
    s1_ref[0] = jnp.sum(jnp.min(d, axis=1)).reshape(1, 1)
    s2_ref[0] = jnp.sum(jnp.min(d, axis=0)).reshape(1, 1)


def _decode_chamfer(z, pos_pad, params):
    G2 = GRID * GRID
    xs = jnp.linspace(-0.3, 0.3, GRID)
    gxm, gym = jnp.meshgrid(xs, xs, indexing='ij')
    gx = gxm.ravel()[:, None].astype(jnp.float32)
    gy = gym.ravel()[:, None].astype(jnp.float32)

    f1w1 = params['f1_w1']
    f2w1 = params['f2_w1']
    f1w3 = jnp.zeros((8, 512), jnp.float32).at[:3].set(params['f1_w3'])
    f1b3 = jnp.zeros((1, 8), jnp.float32).at[0, :3].set(params['f1_b3'])
    f2w3 = jnp.zeros((8, 512), jnp.float32).at[:3].set(params['f2_w3'])
    f2b3 = jnp.zeros((1, 8), jnp.float32).at[0, :3].set(params['f2_b3'])

    bcast = lambda b: (0, 0)
    recon, s1, s2 = pl.pallas_call(
        _dec_kernel,
        grid=(B,),
        in_specs=[
            pl.BlockSpec((1, 1, LATENT), lambda b: (b, 0, 0)),
            pl.BlockSpec((1, 8, P), lambda b: (b, 0, 0)),
            pl.BlockSpec((G2, 1), bcast),
            pl.BlockSpec((G2, 1), bcast),
            pl.BlockSpec((512, LATENT), bcast),
            pl.BlockSpec((2, 512), bcast),
            pl.BlockSpec((1, 512), bcast),
            pl.BlockSpec((512, 512), bcast),
            pl.BlockSpec((1, 512), bcast),
            pl.BlockSpec((8, 512), bcast),
            pl.BlockSpec((1, 8), bcast),
            pl.BlockSpec((512, LATENT), bcast),
            pl.BlockSpec((3, 512), bcast),
            pl.BlockSpec((1, 512), bcast),
            pl.BlockSpec((512, 512), bcast),
            pl.BlockSpec((1, 512), bcast),
            pl.BlockSpec((8, 512), bcast),
            pl.BlockSpec((1, 8), bcast),
        ],
        out_specs=[
            pl.BlockSpec((1, G2, 8), lambda b: (b, 0, 0)),
            pl.BlockSpec((1, 1, 1), lambda b: (b, 0, 0)),
            pl.BlockSpec((1, 1, 1), lambda b: (b, 0, 0)),
        ],
        out_shape=[
            jax.ShapeDtypeStruct((B, G2, 8), jnp.float32),
            jax.ShapeDtypeStruct((B, 1, 1), jnp.float32),
            jax.ShapeDtypeStruct((B, 1, 1), jnp.float32),
        ],
    )(
        z[:, None, :], pos_pad, gx, gy,
        f1w1[:, :LATENT], f1w1[:, LATENT:].T, params['f1_b1'][None, :],
        params['f1_w2'], params['f1_b2'][None, :], f1w3, f1b3,
        f2w1[:, :LATENT], f2w1[:, LATENT:].T, params['f2_b1'][None, :],
        params['f2_w2'], params['f2_b2'][None, :], f2w3, f2b3,
    )
    recon = recon[:, :, :3]
    chamfer = jnp.sum(s1) / (B * G2) + jnp.sum(s2) / (B * P)
    return recon, chamfer


def kernel(pos, params, batch):
    pos_b = pos.reshape(B, P, 3)
    pos_t = jnp.transpose(pos_b, (2, 0, 1))
    q1t, q2t, q3t = _fps_all(pos_t)
    q1 = jnp.transpose(q1t, (1, 0, 2))
    q2 = jnp.transpose(q2t, (1, 0, 2))
    q3 = jnp.transpose(q3t, (1, 0, 2))
    pooled = jax.vmap(_encode_graph, in_axes=(0, 0, 0, 0, None))(
        pos_b, q1, q2, q3, params)
    mu = _linear(pooled, params['mu_w'], params['mu_b'])
    logvar = _linear(pooled, params['lv_w'], params['lv_b'])
    std = jnp.exp(0.5 * logvar)
    eps = jax.random.normal(jax.random.key(42), mu.shape, dtype=mu.dtype)
    z = mu + std * eps

    pos_pad = jnp.zeros((B, 8, P), jnp.float32).at[:, :3, :].set(
        jnp.transpose(pos_b, (0, 2, 1)))
    recon, chamfer = _decode_chamfer(z, pos_pad, params)

    kl = -0.5 * jnp.mean(jnp.sum(1.0 + logvar - mu ** 2 - jnp.exp(logvar), axis=-1))
    loss = chamfer + 0.001 * kl
    return loss, chamfer, kl, mu, recon
